# baseline (speedup 1.0000x reference)
import jax
import jax.numpy as jnp
from jax.experimental import pallas as pl
from jax.experimental.pallas import tpu as pltpu

_S = 0.1
_R2 = 0.02
_EPS = 1e-12


def _scalar_chain(t, zz, n):
    beta1 = (jnp.clip(t, -_S, _S) - t) * (1.0 / n)
    ok1 = zz + (2.0 * t + n * beta1) * beta1 <= _R2
    scale = jnp.minimum(1.0, jnp.sqrt(_R2) * jax.lax.rsqrt(jnp.maximum(zz, _EPS)))
    ok2 = jnp.abs(t) * scale <= _S
    denom = jnp.maximum(n * zz - t * t, _EPS)
    c = jnp.sqrt(jnp.maximum(n * _R2 - _S * _S, 0.0)) * jax.lax.rsqrt(denom)
    beta3 = (jnp.sign(t) * _S - c * t) * (1.0 / n)
    alpha = jnp.where(ok1, 1.0, jnp.where(ok2, scale, c))
    beta = jnp.where(ok1, beta1, jnp.where(ok2, 0.0, beta3))
    return alpha, beta


def _body(x_ref, w_ref, b_ref, o_ref, acc_ref, *, n_k):
    k = pl.program_id(1)
    zk = jax.lax.dot_general(
        w_ref[...], x_ref[...],
        dimension_numbers=(((1,), (1,)), ((), ())),
        preferred_element_type=jnp.float32,
    )

    @pl.when(k == 0)
    def _():
        acc_ref[...] = zk

    @pl.when(k != 0)
    def _():
        acc_ref[...] += zk

    @pl.when(k == n_k - 1)
    def _():
        zt = acc_ref[...] - b_ref[...]
        n = zt.shape[0]
        t = jnp.sum(zt, axis=0, keepdims=True)
        zz = jnp.sum(zt * zt, axis=0, keepdims=True)
        alpha, beta = _scalar_chain(t, zz, n)
        o_ref[...] = (alpha * zt + beta).T


def kernel(x, W, b):
    B, D_in = x.shape
    D_out = W.shape[0]
    BT = 8192
    NK = 2
    KC = D_in // NK
    b2 = b.reshape(D_out, 1)
    import functools
    body = functools.partial(_body, n_k=NK)
    return pl.pallas_call(
        body,
        grid=(B // BT, NK),
        in_specs=[
            pl.BlockSpec((BT, KC), lambda i, k: (i, k)),
            pl.BlockSpec((D_out, KC), lambda i, k: (0, k)),
            pl.BlockSpec((D_out, 1), lambda i, k: (0, 0)),
        ],
        out_specs=pl.BlockSpec((BT, D_out), lambda i, k: (i, 0)),
        out_shape=jax.ShapeDtypeStruct((B, D_out), jnp.float32),
        scratch_shapes=[pltpu.VMEM((D_out, BT), jnp.float32)],
        compiler_params=pltpu.CompilerParams(
            dimension_semantics=("arbitrary", "arbitrary"),
        ),
        name="optlayer_fused",
    )(x, W, b2)


# speedup vs baseline: 1.4552x; 1.4552x over previous
import jax
import jax.numpy as jnp
from jax.experimental import pallas as pl
from jax.experimental.pallas import tpu as pltpu

_S = 0.1
_R2 = 0.02
_EPS = 1e-12


def _scalar_chain(t, zz, n):
    beta1 = (jnp.clip(t, -_S, _S) - t) * (1.0 / n)
    ok1 = zz + (2.0 * t + n * beta1) * beta1 <= _R2
    scale = jnp.minimum(1.0, jnp.sqrt(_R2) * jax.lax.rsqrt(jnp.maximum(zz, _EPS)))
    ok2 = jnp.abs(t) * scale <= _S
    denom = jnp.maximum(n * zz - t * t, _EPS)
    c = jnp.sqrt(jnp.maximum(n * _R2 - _S * _S, 0.0)) * jax.lax.rsqrt(denom)
    beta3 = (jnp.sign(t) * _S - c * t) * (1.0 / n)
    alpha = jnp.where(ok1, 1.0, jnp.where(ok2, scale, c))
    beta = jnp.where(ok1, beta1, jnp.where(ok2, 0.0, beta3))
    return alpha, beta


def _body(x_ref, w_ref, b_ref, o_ref):
    zt = jax.lax.dot_general(
        w_ref[...], x_ref[...],
        dimension_numbers=(((1,), (1,)), ((), ())),
        preferred_element_type=jnp.float32,
    )
    zt = zt - b_ref[...]
    n = zt.shape[0]
    t = jnp.sum(zt, axis=0, keepdims=True)
    zz = jnp.sum(zt * zt, axis=0, keepdims=True)
    alpha, beta = _scalar_chain(t, zz, n)
    o_ref[...] = (alpha * zt + beta).T


def kernel(x, W, b):
    B, D_in = x.shape
    D_out = W.shape[0]
    BT = 8192
    b2 = b.reshape(D_out, 1)
    return pl.pallas_call(
        _body,
        grid=(B // BT,),
        in_specs=[
            pl.BlockSpec((BT, D_in), lambda i: (i, 0)),
            pl.BlockSpec((D_out, D_in), lambda i: (0, 0)),
            pl.BlockSpec((D_out, 1), lambda i: (0, 0)),
        ],
        out_specs=pl.BlockSpec((BT, D_out), lambda i: (i, 0)),
        out_shape=jax.ShapeDtypeStruct((B, D_out), jnp.float32),
        compiler_params=pltpu.CompilerParams(
            dimension_semantics=("arbitrary",),
        ),
        name="optlayer_fused",
    )(x, W, b2)
